# baseline (speedup 1.0000x reference)
import jax
import jax.numpy as jnp
from jax.experimental import pallas as pl

_TOPK = 2
_BM = 512


def _router_kernel(x_ref, w_ref, idx_ref, gate_ref):
    x = x_ref[...]
    w = w_ref[...]
    logits = jax.lax.dot_general(
        w, x, (((1,), (1,)), ((), ())),
        preferred_element_type=jnp.float32)
    e = logits.shape[0]
    row = jax.lax.broadcasted_iota(jnp.int32, logits.shape, 0)

    l1 = jnp.max(logits, axis=0, keepdims=True)
    i1 = jnp.min(jnp.where(logits == l1, row, e), axis=0, keepdims=True)
    masked = jnp.where(row == i1, -jnp.inf, logits)
    l2 = jnp.max(masked, axis=0, keepdims=True)
    i2 = jnp.min(jnp.where(masked == l2, row, e), axis=0, keepdims=True)

    e2 = jnp.exp(l2 - l1)
    denom = 1.0 + e2
    idx_ref[...] = jnp.concatenate([i1, i2], axis=0)
    gate_ref[...] = jnp.concatenate([1.0 / denom, e2 / denom], axis=0)


@jax.jit
def kernel(x, W):
    m, k = x.shape
    e = W.shape[0]
    grid = (m // _BM,)
    idx_t, gates_t = pl.pallas_call(
        _router_kernel,
        grid=grid,
        in_specs=[
            pl.BlockSpec((_BM, k), lambda i: (i, 0)),
            pl.BlockSpec((e, k), lambda i: (0, 0)),
        ],
        out_specs=[
            pl.BlockSpec((_TOPK, _BM), lambda i: (0, i)),
            pl.BlockSpec((_TOPK, _BM), lambda i: (0, i)),
        ],
        out_shape=[
            jax.ShapeDtypeStruct((_TOPK, m), jnp.int32),
            jax.ShapeDtypeStruct((_TOPK, m), jnp.float32),
        ],
    )(x, W)
    return idx_t.T, gates_t.T


# speedup vs baseline: 1.2034x; 1.2034x over previous
import jax
import jax.numpy as jnp
from jax.experimental import pallas as pl

_TOPK = 2
_BM = 1024


def _router_kernel(x_ref, w_ref, idx_ref, gate_ref):
    x = x_ref[...]
    w = w_ref[...]
    logits = jax.lax.dot_general(
        w, x, (((1,), (1,)), ((), ())),
        preferred_element_type=jnp.float32)
    e = logits.shape[0]
    row = jax.lax.broadcasted_iota(jnp.int32, logits.shape, 0)

    l1 = jnp.max(logits, axis=0, keepdims=True)
    i1 = jnp.min(jnp.where(logits == l1, row, e), axis=0, keepdims=True)
    masked = jnp.where(row == i1, -jnp.inf, logits)
    l2 = jnp.max(masked, axis=0, keepdims=True)
    i2 = jnp.min(jnp.where(masked == l2, row, e), axis=0, keepdims=True)

    e2 = jnp.exp(l2 - l1)
    denom = 1.0 + e2
    idx_ref[...] = jnp.concatenate([i1, i2], axis=0)
    gate_ref[...] = jnp.concatenate([1.0 / denom, e2 / denom], axis=0)


@jax.jit
def kernel(x, W):
    m, k = x.shape
    e = W.shape[0]
    grid = (m // _BM,)
    idx_t, gates_t = pl.pallas_call(
        _router_kernel,
        grid=grid,
        in_specs=[
            pl.BlockSpec((_BM, k), lambda i: (i, 0)),
            pl.BlockSpec((e, k), lambda i: (0, 0)),
        ],
        out_specs=[
            pl.BlockSpec((_TOPK, _BM), lambda i: (0, i)),
            pl.BlockSpec((_TOPK, _BM), lambda i: (0, i)),
        ],
        out_shape=[
            jax.ShapeDtypeStruct((_TOPK, m), jnp.int32),
            jax.ShapeDtypeStruct((_TOPK, m), jnp.float32),
        ],
    )(x, W)
    return idx_t.T, gates_t.T
